# Initial kernel scaffold; baseline (speedup 1.0000x reference)
import jax
import jax.numpy as jnp
from jax.experimental import pallas as pl


def kernel(x, edge_index_int, edge_index_nh, W1_int, b1_int, W1_nh, b1_nh, W2_int, b2_int, W2_nh, b2_nh, att_w, dense_W, dense_b):
    raise NotImplementedError("write your pallas kernel here")


# speedup vs baseline: 18.3572x; 18.3572x over previous
import jax
import jax.numpy as jnp
from jax import lax
from jax.experimental import pallas as pl
from jax.experimental.pallas import tpu as pltpu
from jax.experimental.pallas import tpu_sc as plsc

N = 100000
E = 1600000
D = 16
NC = 2
NS = 16
ET = E // NS
W = 1000
NWIN = ET // W
N_ACC = 100096
RT = N_ACC // NS


def _seg_sum_body(table, src, dst, zeros, out, acc, src_v, dst_v, rows_v, sem):
    cid = lax.axis_index("c")
    sid = lax.axis_index("s")

    r0 = pl.multiple_of(sid * RT, 8)
    pltpu.sync_copy(zeros.at[pl.ds(r0, RT)], acc.at[pl.ds(r0, RT)])
    plsc.subcore_barrier()

    base = cid * E + sid * ET

    def step(w, carry):
        off = pl.multiple_of(base + w * W, 8)
        pltpu.sync_copy(src.at[pl.ds(off, W)], src_v)
        pltpu.sync_copy(dst.at[pl.ds(off, W)], dst_v)
        pltpu.async_copy(table.at[src_v], rows_v, sem).wait()
        pltpu.sync_copy(rows_v, acc.at[dst_v], add=True)
        return carry

    lax.fori_loop(0, NWIN, step, 0)
    plsc.subcore_barrier()
    o0 = pl.multiple_of(cid * N_ACC + sid * RT, 8)
    pltpu.sync_copy(acc.at[pl.ds(r0, RT)], out.at[pl.ds(o0, RT)])


def _sc_seg_sum(table, src, dst, zeros):
    mesh = plsc.VectorSubcoreMesh(core_axis_name="c", subcore_axis_name="s")
    return pl.kernel(
        _seg_sum_body,
        out_type=jax.ShapeDtypeStruct((2 * N_ACC, D), jnp.float32),
        mesh=mesh,
        scratch_types=[
            pltpu.VMEM_SHARED((N_ACC, D), jnp.float32),
            pltpu.VMEM((W,), jnp.int32),
            pltpu.VMEM((W,), jnp.int32),
            pltpu.VMEM((W, D), jnp.float32),
            pltpu.SemaphoreType.DMA,
        ],
        compiler_params=pltpu.CompilerParams(use_tc_tiling_on_sc=False),
    )(table, src, dst, zeros)


BLK = 6256
NBLK = 2 * N_ACC // BLK
HALF = NBLK // 2


def _mm_relu_body(a_ref, w_ref, b_ref, o_ref):
    o_ref[...] = jax.nn.relu(
        jnp.dot(a_ref[...], w_ref[0], preferred_element_type=jnp.float32,
                precision=jax.lax.Precision.HIGHEST)
        + b_ref[0])


def _tc_mm_relu(agg, w_stacked, b_stacked):
    return pl.pallas_call(
        _mm_relu_body,
        grid=(NBLK,),
        in_specs=[
            pl.BlockSpec((BLK, 16), lambda p: (p, 0)),
            pl.BlockSpec((1, 16, 16), lambda p: (p // HALF, 0, 0)),
            pl.BlockSpec((1, 1, 16), lambda p: (p // HALF, 0, 0)),
        ],
        out_specs=pl.BlockSpec((BLK, 16), lambda p: (p, 0)),
        out_shape=jax.ShapeDtypeStruct((2 * N_ACC, 16), jnp.float32),
    )(agg, w_stacked, b_stacked)


BLK2 = 2000
NBLK2 = N // BLK2


def _final_body(a_int_ref, a_nh_ref, w2i_ref, b2i_ref, w2n_ref, b2n_ref,
                att_ref, dwr_ref, db_ref, o_ref, l_ref, acc_ref):
    p = pl.program_id(0)

    @pl.when(p == 0)
    def _():
        l_ref[...] = jnp.zeros_like(l_ref)
        acc_ref[...] = jnp.zeros_like(acc_ref)

    h_int = jax.nn.relu(
        jnp.dot(a_int_ref[0], w2i_ref[...], preferred_element_type=jnp.float32,
                precision=jax.lax.Precision.HIGHEST)
        + b2i_ref[...])
    h_nh = jax.nn.relu(
        jnp.dot(a_nh_ref[0], w2n_ref[...], preferred_element_type=jnp.float32,
                precision=jax.lax.Precision.HIGHEST)
        + b2n_ref[...])
    h = jnp.concatenate([h_int, h_nh], axis=1)
    s = jnp.tanh(jnp.dot(h, att_ref[...],
                         preferred_element_type=jnp.float32,
                precision=jax.lax.Precision.HIGHEST))
    e = jnp.exp(s)
    l_ref[0:1, 0:3] += jnp.sum(e, axis=0, keepdims=True)
    for hh in range(3):
        acc_ref[hh:hh + 1, 0:64] += jnp.sum(
            h * e[:, hh:hh + 1], axis=0, keepdims=True)

    @pl.when(p == NBLK2 - 1)
    def _():
        acc = acc_ref[0:3, 0:64]
        lsum = l_ref[0:1, 0:3]
        per_head = jnp.sum(acc * dwr_ref[...], axis=1)
        o_ref[...] = jnp.sum(per_head / lsum[0]).reshape(1, 1) + db_ref[...]


def _tc_final(agg2, W2_int, b2_int, W2_nh, b2_nh, att_w, dwr, db):
    a3 = agg2.reshape(2, N_ACC, 16)
    return pl.pallas_call(
        _final_body,
        grid=(NBLK2,),
        in_specs=[
            pl.BlockSpec((1, BLK2, 16), lambda p: (0, p, 0)),
            pl.BlockSpec((1, BLK2, 16), lambda p: (1, p, 0)),
            pl.BlockSpec((16, 32), lambda p: (0, 0)),
            pl.BlockSpec((1, 32), lambda p: (0, 0)),
            pl.BlockSpec((16, 32), lambda p: (0, 0)),
            pl.BlockSpec((1, 32), lambda p: (0, 0)),
            pl.BlockSpec((64, 3), lambda p: (0, 0)),
            pl.BlockSpec((3, 64), lambda p: (0, 0)),
            pl.BlockSpec((1, 1), lambda p: (0, 0)),
        ],
        out_specs=pl.BlockSpec((1, 1), lambda p: (0, 0)),
        out_shape=jax.ShapeDtypeStruct((1, 1), jnp.float32),
        scratch_shapes=[
            pltpu.VMEM((8, 128), jnp.float32),
            pltpu.VMEM((8, 128), jnp.float32),
        ],
    )(a3, a3, W2_int, b2_int.reshape(1, 32), W2_nh, b2_nh.reshape(1, 32),
      att_w, dwr, db)


@jax.jit
def kernel(x, edge_index_int, edge_index_nh, W1_int, b1_int, W1_nh, b1_nh,
           W2_int, b2_int, W2_nh, b2_nh, att_w, dense_W, dense_b):
    x_pad = jnp.pad(x, ((0, 0), (0, D - x.shape[1])))
    rowpad = jnp.zeros((N_ACC - N, D), jnp.float32)
    table1 = jnp.concatenate([x_pad, rowpad, x_pad, rowpad], axis=0)
    src_cat = jnp.concatenate(
        [edge_index_int[0], edge_index_nh[0] + N_ACC])
    dst_cat = jnp.concatenate([edge_index_int[1], edge_index_nh[1]])
    zeros = jnp.zeros((N_ACC, D), jnp.float32)
    W1p = jnp.stack([
        jnp.zeros((16, 16), jnp.float32).at[:11].set(W1_int),
        jnp.zeros((16, 16), jnp.float32).at[:11].set(W1_nh),
    ])
    b1s = jnp.stack([b1_int, b1_nh]).reshape(2, 1, 16)

    agg1 = _sc_seg_sum(table1, src_cat, dst_cat, zeros)
    h1 = _tc_mm_relu(agg1, W1p, b1s)

    agg2 = _sc_seg_sum(h1, src_cat, dst_cat, zeros)

    out = _tc_final(agg2, W2_int, b2_int, W2_nh, b2_nh, att_w,
                    dense_W.reshape(3, 64), dense_b.reshape(1, 1))
    return out.reshape(())


# speedup vs baseline: 36.3373x; 1.9795x over previous
import jax
import jax.numpy as jnp
from jax import lax
from jax.experimental import pallas as pl
from jax.experimental.pallas import tpu as pltpu
from jax.experimental.pallas import tpu_sc as plsc

N = 100000
E = 1600000
D = 16
NS = 16
ET = E // NS
W = 800
NWIN = ET // W
N_ACC = 100352
RT = N_ACC // NS
PK = N_ACC // 8
PKN = N // 8


def _core_seg_sum(sid, table, src, dst, out, acc,
                  src_a, dst_a, rows_a, src_b, dst_b, rows_b,
                  sem_a, sem_b, sem_sa, sem_sb):
    base = sid * ET

    def prime(w, src_v, dst_v, rows_v, sem):
        off = pl.multiple_of(base + w * W, 8)
        pltpu.sync_copy(src.at[pl.ds(off, W)], src_v)
        pltpu.sync_copy(dst.at[pl.ds(off, W)], dst_v)
        pltpu.async_copy(table.at[src_v], rows_v, sem)

    def drain(src_v, dst_v, rows_v, sem, sem_s):
        pltpu.make_async_copy(table.at[src_v], rows_v, sem).wait()
        pltpu.async_copy(rows_v, acc.at[dst_v], sem_s, add=True)
        pltpu.make_async_copy(rows_v, acc.at[dst_v], sem_s).wait()

    prime(0, src_a, dst_a, rows_a, sem_a)

    def step(k, carry):
        prime(2 * k + 1, src_b, dst_b, rows_b, sem_b)
        drain(src_a, dst_a, rows_a, sem_a, sem_sa)
        prime(2 * k + 2, src_a, dst_a, rows_a, sem_a)
        drain(src_b, dst_b, rows_b, sem_b, sem_sb)
        return carry

    lax.fori_loop(0, NWIN // 2, step, 0)
    drain(src_a, dst_a, rows_a, sem_a, sem_sa)
    plsc.subcore_barrier()
    r0 = pl.multiple_of(sid * RT, 8)
    pltpu.sync_copy(acc.at[pl.ds(r0, RT)], out.at[pl.ds(r0, RT)])


def _seg_sum_body(table_int, table_nh, src_int, dst_int, src_nh, dst_nh,
                  zeros, out_int, out_nh, acc,
                  src_a, dst_a, rows_a, src_b, dst_b, rows_b,
                  sem_a, sem_b, sem_sa, sem_sb):
    cid = lax.axis_index("c")
    sid = lax.axis_index("s")

    r0 = pl.multiple_of(sid * RT, 8)
    pltpu.sync_copy(zeros.at[pl.ds(r0, RT)], acc.at[pl.ds(r0, RT)])
    plsc.subcore_barrier()

    @pl.when(cid == 0)
    def _():
        _core_seg_sum(sid, table_int, src_int, dst_int, out_int, acc,
                      src_a, dst_a, rows_a, src_b, dst_b, rows_b,
                      sem_a, sem_b, sem_sa, sem_sb)

    @pl.when(cid == 1)
    def _():
        _core_seg_sum(sid, table_nh, src_nh, dst_nh, out_nh, acc,
                      src_a, dst_a, rows_a, src_b, dst_b, rows_b,
                      sem_a, sem_b, sem_sa, sem_sb)


def _sc_seg_sum(table_int, table_nh, src_int, dst_int, src_nh, dst_nh, zeros):
    mesh = plsc.VectorSubcoreMesh(core_axis_name="c", subcore_axis_name="s")
    return pl.kernel(
        _seg_sum_body,
        out_type=(jax.ShapeDtypeStruct((N_ACC, D), jnp.float32),
                  jax.ShapeDtypeStruct((N_ACC, D), jnp.float32)),
        mesh=mesh,
        scratch_types=[
            pltpu.VMEM_SHARED((N_ACC, D), jnp.float32),
            pltpu.VMEM((W,), jnp.int32),
            pltpu.VMEM((W,), jnp.int32),
            pltpu.VMEM((W, D), jnp.float32),
            pltpu.VMEM((W,), jnp.int32),
            pltpu.VMEM((W,), jnp.int32),
            pltpu.VMEM((W, D), jnp.float32),
            pltpu.SemaphoreType.DMA,
            pltpu.SemaphoreType.DMA,
            pltpu.SemaphoreType.DMA,
            pltpu.SemaphoreType.DMA,
        ],
        compiler_params=pltpu.CompilerParams(use_tc_tiling_on_sc=False),
    )(table_int, table_nh, src_int, dst_int, src_nh, dst_nh, zeros)


BLK = 1568
NBLK = PK // BLK


def _mm_relu_body(a1_ref, a2_ref, w1_ref, b1_ref, w2_ref, b2_ref,
                  o1_ref, o2_ref):
    o1_ref[...] = jax.nn.relu(
        jnp.dot(a1_ref[...], w1_ref[...], preferred_element_type=jnp.float32,
                precision=jax.lax.Precision.HIGHEST) + b1_ref[...])
    o2_ref[...] = jax.nn.relu(
        jnp.dot(a2_ref[...], w2_ref[...], preferred_element_type=jnp.float32,
                precision=jax.lax.Precision.HIGHEST) + b2_ref[...])


def _tc_mm_relu(a1, a2, w1b, b1t, w2b, b2t):
    return pl.pallas_call(
        _mm_relu_body,
        grid=(NBLK,),
        in_specs=[
            pl.BlockSpec((BLK, 128), lambda p: (p, 0)),
            pl.BlockSpec((BLK, 128), lambda p: (p, 0)),
            pl.BlockSpec((128, 128), lambda p: (0, 0)),
            pl.BlockSpec((1, 128), lambda p: (0, 0)),
            pl.BlockSpec((128, 128), lambda p: (0, 0)),
            pl.BlockSpec((1, 128), lambda p: (0, 0)),
        ],
        out_specs=[
            pl.BlockSpec((BLK, 128), lambda p: (p, 0)),
            pl.BlockSpec((BLK, 128), lambda p: (p, 0)),
        ],
        out_shape=[jax.ShapeDtypeStruct((PK, 128), jnp.float32),
                   jax.ShapeDtypeStruct((PK, 128), jnp.float32)],
    )(a1, a2, w1b, b1t, w2b, b2t)


BLK2 = 1568
NBLK2 = PK // BLK2


def _final_body(ai_ref, an_ref, w2i_ref, b2i_ref, w2n_ref, b2n_ref,
                abi_ref, abn_ref, dbi_ref, dbn_ref, db_ref,
                o_ref, acc_ref):
    p = pl.program_id(0)

    @pl.when(p == 0)
    def _():
        acc_ref[...] = jnp.zeros_like(acc_ref)

    hp = jax.lax.Precision.HIGHEST
    h_i = jax.nn.relu(
        jnp.dot(ai_ref[...], w2i_ref[...], preferred_element_type=jnp.float32,
                precision=hp) + b2i_ref[...])
    h_n = jax.nn.relu(
        jnp.dot(an_ref[...], w2n_ref[...], preferred_element_type=jnp.float32,
                precision=hp) + b2n_ref[...])
    s = jnp.tanh(
        jnp.dot(h_i, abi_ref[...], preferred_element_type=jnp.float32,
                precision=hp)
        + jnp.dot(h_n, abn_ref[...], preferred_element_type=jnp.float32,
                  precision=hp))
    e = jnp.exp(s)
    proj = (jnp.dot(h_i, dbi_ref[...], preferred_element_type=jnp.float32,
                    precision=hp)
            + jnp.dot(h_n, dbn_ref[...], preferred_element_type=jnp.float32,
                      precision=hp))
    row = p * BLK2 + jax.lax.broadcasted_iota(jnp.int32, (BLK2, 1), 0)
    e = jnp.where(row < PKN, e, 0.0)
    acc_ref[0:1, 0:24] += jnp.sum(e, axis=0, keepdims=True)
    acc_ref[1:2, 0:24] += jnp.sum(e * proj, axis=0, keepdims=True)

    @pl.when(p == NBLK2 - 1)
    def _():
        den = jnp.zeros((1, 3), jnp.float32)
        num = jnp.zeros((1, 3), jnp.float32)
        for g in range(8):
            den += acc_ref[0:1, 3 * g:3 * g + 3]
            num += acc_ref[1:2, 3 * g:3 * g + 3]
        o_ref[...] = jnp.sum(num / den).reshape(1, 1) + db_ref[...]


def _tc_final(ai, an, w2bi, b2ti, w2bn, b2tn, abi, abn, dbi, dbn, db):
    return pl.pallas_call(
        _final_body,
        grid=(NBLK2,),
        in_specs=[
            pl.BlockSpec((BLK2, 128), lambda p: (p, 0)),
            pl.BlockSpec((BLK2, 128), lambda p: (p, 0)),
            pl.BlockSpec((128, 256), lambda p: (0, 0)),
            pl.BlockSpec((1, 256), lambda p: (0, 0)),
            pl.BlockSpec((128, 256), lambda p: (0, 0)),
            pl.BlockSpec((1, 256), lambda p: (0, 0)),
            pl.BlockSpec((256, 24), lambda p: (0, 0)),
            pl.BlockSpec((256, 24), lambda p: (0, 0)),
            pl.BlockSpec((256, 24), lambda p: (0, 0)),
            pl.BlockSpec((256, 24), lambda p: (0, 0)),
            pl.BlockSpec((1, 1), lambda p: (0, 0)),
        ],
        out_specs=pl.BlockSpec((1, 1), lambda p: (0, 0)),
        out_shape=jax.ShapeDtypeStruct((1, 1), jnp.float32),
        scratch_shapes=[pltpu.VMEM((8, 128), jnp.float32)],
    )(ai, an, w2bi, b2ti, w2bn, b2tn, abi, abn, dbi, dbn, db)


def _blockdiag(w):
    return jnp.kron(jnp.eye(8, dtype=jnp.float32), w)


@jax.jit
def kernel(x, edge_index_int, edge_index_nh, W1_int, b1_int, W1_nh, b1_nh,
           W2_int, b2_int, W2_nh, b2_nh, att_w, dense_W, dense_b):
    x_pad = jnp.pad(x, ((0, N_ACC - N), (0, D - x.shape[1])))
    zeros = jnp.zeros((N_ACC, D), jnp.float32)
    w1bi = _blockdiag(jnp.zeros((16, 16), jnp.float32).at[:11].set(W1_int))
    w1bn = _blockdiag(jnp.zeros((16, 16), jnp.float32).at[:11].set(W1_nh))
    b1ti = jnp.tile(b1_int, 8).reshape(1, 128)
    b1tn = jnp.tile(b1_nh, 8).reshape(1, 128)
    w2bi = _blockdiag(W2_int)
    w2bn = _blockdiag(W2_nh)
    b2ti = jnp.tile(b2_int, 8).reshape(1, 256)
    b2tn = jnp.tile(b2_nh, 8).reshape(1, 256)
    abi = _blockdiag(att_w[:32])
    abn = _blockdiag(att_w[32:])
    dwr = dense_W.reshape(3, 64)
    dbi = _blockdiag(dwr[:, :32].T)
    dbn = _blockdiag(dwr[:, 32:].T)
    db = dense_b.reshape(1, 1)

    agg1_i, agg1_n = _sc_seg_sum(
        x_pad, x_pad, edge_index_int[0], edge_index_int[1],
        edge_index_nh[0], edge_index_nh[1], zeros)

    h1_i, h1_n = _tc_mm_relu(agg1_i.reshape(PK, 128), agg1_n.reshape(PK, 128),
                             w1bi, b1ti, w1bn, b1tn)

    agg2_i, agg2_n = _sc_seg_sum(
        h1_i.reshape(N_ACC, D), h1_n.reshape(N_ACC, D),
        edge_index_int[0], edge_index_int[1],
        edge_index_nh[0], edge_index_nh[1], zeros)

    out = _tc_final(agg2_i.reshape(PK, 128), agg2_n.reshape(PK, 128),
                    w2bi, b2ti, w2bn, b2tn, abi, abn, dbi, dbn, db)
    return out.reshape(())


# speedup vs baseline: 40.0531x; 1.1023x over previous
import jax
import jax.numpy as jnp
from jax import lax
from jax.experimental import pallas as pl
from jax.experimental.pallas import tpu as pltpu
from jax.experimental.pallas import tpu_sc as plsc

N = 100000
E = 1600000
D = 16
NS = 16
ET = E // NS
W = 400
NWIN = ET // W
N_ACC = 100096
RT = N_ACC // NS
PK = N_ACC // 8
PKN = N // 8


def _core_seg_sum(sid, table, src, dst, out, acc, bufs):
    base = sid * ET
    srcs, dsts, rows, sem_i, sem_g, sem_s = bufs

    def idx_start(w, s):
        off = pl.multiple_of(base + w * W, 8)
        pltpu.async_copy(src.at[pl.ds(off, W)], srcs[s], sem_i[s])
        pltpu.async_copy(dst.at[pl.ds(off, W)], dsts[s], sem_i[s])

    def gather_start(s):
        pltpu.make_async_copy(src.at[pl.ds(0, W)], srcs[s], sem_i[s]).wait()
        pltpu.make_async_copy(dst.at[pl.ds(0, W)], dsts[s], sem_i[s]).wait()
        pltpu.async_copy(table.at[srcs[s]], rows[s], sem_g[s])

    def scatter_start(s):
        pltpu.make_async_copy(table.at[srcs[s]], rows[s], sem_g[s]).wait()
        pltpu.async_copy(rows[s], acc.at[dsts[s]], sem_s[s], add=True)

    def scatter_wait(s):
        pltpu.make_async_copy(rows[s], acc.at[dsts[s]], sem_s[s]).wait()

    idx_start(0, 0); gather_start(0)
    idx_start(1, 1); gather_start(1)
    idx_start(2, 2); scatter_start(0); gather_start(2)

    def step(k, carry):
        for j in range(3):
            w = 3 * k + j
            scatter_wait(j)
            idx_start(w, j)
            scatter_start((j + 1) % 3)
            gather_start(j)
        return carry

    lax.fori_loop(1, NWIN // 3, step, 0)
    scatter_wait(0); idx_start(NWIN - 1, 0)
    scatter_start(1); gather_start(0)
    scatter_wait(1); scatter_start(2)
    scatter_wait(2); scatter_start(0)
    scatter_wait(0)
    plsc.subcore_barrier()
    r0 = pl.multiple_of(sid * RT, 8)
    pltpu.sync_copy(acc.at[pl.ds(r0, RT)], out.at[pl.ds(r0, RT)])


def _seg_sum_body(table_int, table_nh, src_int, dst_int, src_nh, dst_nh,
                  zeros, out_int, out_nh, acc,
                  s0, d0, r0b, s1, d1, r1b, s2, d2, r2b,
                  si0, si1, si2, sg0, sg1, sg2, ss0, ss1, ss2):
    cid = lax.axis_index("c")
    sid = lax.axis_index("s")
    bufs = ([s0, s1, s2], [d0, d1, d2], [r0b, r1b, r2b],
            [si0, si1, si2], [sg0, sg1, sg2], [ss0, ss1, ss2])

    r0 = pl.multiple_of(sid * RT, 8)
    pltpu.sync_copy(zeros.at[pl.ds(r0, RT)], acc.at[pl.ds(r0, RT)])
    plsc.subcore_barrier()

    @pl.when(cid == 0)
    def _():
        _core_seg_sum(sid, table_int, src_int, dst_int, out_int, acc, bufs)

    @pl.when(cid == 1)
    def _():
        _core_seg_sum(sid, table_nh, src_nh, dst_nh, out_nh, acc, bufs)


def _sc_seg_sum(table_int, table_nh, src_int, dst_int, src_nh, dst_nh, zeros):
    mesh = plsc.VectorSubcoreMesh(core_axis_name="c", subcore_axis_name="s")
    return pl.kernel(
        _seg_sum_body,
        out_type=(jax.ShapeDtypeStruct((N_ACC, D), jnp.float32),
                  jax.ShapeDtypeStruct((N_ACC, D), jnp.float32)),
        mesh=mesh,
        scratch_types=(
            [pltpu.VMEM_SHARED((N_ACC, D), jnp.float32)]
            + [pltpu.VMEM((W,), jnp.int32), pltpu.VMEM((W,), jnp.int32),
               pltpu.VMEM((W, D), jnp.float32)] * 3
            + [pltpu.SemaphoreType.DMA] * 9
        ),
        compiler_params=pltpu.CompilerParams(use_tc_tiling_on_sc=False),
    )(table_int, table_nh, src_int, dst_int, src_nh, dst_nh, zeros)


BLK = 3128
NBLK = PK // BLK


def _mm_relu_body(a1_ref, a2_ref, w1_ref, b1_ref, w2_ref, b2_ref,
                  o1_ref, o2_ref):
    o1_ref[...] = jax.nn.relu(
        jnp.dot(a1_ref[...], w1_ref[...], preferred_element_type=jnp.float32,
                precision=jax.lax.Precision.HIGHEST) + b1_ref[...])
    o2_ref[...] = jax.nn.relu(
        jnp.dot(a2_ref[...], w2_ref[...], preferred_element_type=jnp.float32,
                precision=jax.lax.Precision.HIGHEST) + b2_ref[...])


def _tc_mm_relu(a1, a2, w1b, b1t, w2b, b2t):
    return pl.pallas_call(
        _mm_relu_body,
        grid=(NBLK,),
        in_specs=[
            pl.BlockSpec((BLK, 128), lambda p: (p, 0)),
            pl.BlockSpec((BLK, 128), lambda p: (p, 0)),
            pl.BlockSpec((128, 128), lambda p: (0, 0)),
            pl.BlockSpec((1, 128), lambda p: (0, 0)),
            pl.BlockSpec((128, 128), lambda p: (0, 0)),
            pl.BlockSpec((1, 128), lambda p: (0, 0)),
        ],
        out_specs=[
            pl.BlockSpec((BLK, 128), lambda p: (p, 0)),
            pl.BlockSpec((BLK, 128), lambda p: (p, 0)),
        ],
        out_shape=[jax.ShapeDtypeStruct((PK, 128), jnp.float32),
                   jax.ShapeDtypeStruct((PK, 128), jnp.float32)],
    )(a1, a2, w1b, b1t, w2b, b2t)


BLK2 = 3128
NBLK2 = PK // BLK2


def _final_body(ai_ref, an_ref, w2i_ref, b2i_ref, w2n_ref, b2n_ref,
                abi_ref, abn_ref, dbi_ref, dbn_ref, db_ref,
                o_ref, acc_ref):
    p = pl.program_id(0)

    @pl.when(p == 0)
    def _():
        acc_ref[...] = jnp.zeros_like(acc_ref)

    hp = jax.lax.Precision.HIGHEST
    h_i = jax.nn.relu(
        jnp.dot(ai_ref[...], w2i_ref[...], preferred_element_type=jnp.float32,
                precision=hp) + b2i_ref[...])
    h_n = jax.nn.relu(
        jnp.dot(an_ref[...], w2n_ref[...], preferred_element_type=jnp.float32,
                precision=hp) + b2n_ref[...])
    s = jnp.tanh(
        jnp.dot(h_i, abi_ref[...], preferred_element_type=jnp.float32,
                precision=hp)
        + jnp.dot(h_n, abn_ref[...], preferred_element_type=jnp.float32,
                  precision=hp))
    e = jnp.exp(s)
    proj = (jnp.dot(h_i, dbi_ref[...], preferred_element_type=jnp.float32,
                    precision=hp)
            + jnp.dot(h_n, dbn_ref[...], preferred_element_type=jnp.float32,
                      precision=hp))
    row = p * BLK2 + jax.lax.broadcasted_iota(jnp.int32, (BLK2, 1), 0)
    e = jnp.where(row < PKN, e, 0.0)
    acc_ref[0:1, 0:24] += jnp.sum(e, axis=0, keepdims=True)
    acc_ref[1:2, 0:24] += jnp.sum(e * proj, axis=0, keepdims=True)

    @pl.when(p == NBLK2 - 1)
    def _():
        den = jnp.zeros((1, 3), jnp.float32)
        num = jnp.zeros((1, 3), jnp.float32)
        for g in range(8):
            den += acc_ref[0:1, 3 * g:3 * g + 3]
            num += acc_ref[1:2, 3 * g:3 * g + 3]
        o_ref[...] = jnp.sum(num / den).reshape(1, 1) + db_ref[...]


def _tc_final(ai, an, w2bi, b2ti, w2bn, b2tn, abi, abn, dbi, dbn, db):
    return pl.pallas_call(
        _final_body,
        grid=(NBLK2,),
        in_specs=[
            pl.BlockSpec((BLK2, 128), lambda p: (p, 0)),
            pl.BlockSpec((BLK2, 128), lambda p: (p, 0)),
            pl.BlockSpec((128, 256), lambda p: (0, 0)),
            pl.BlockSpec((1, 256), lambda p: (0, 0)),
            pl.BlockSpec((128, 256), lambda p: (0, 0)),
            pl.BlockSpec((1, 256), lambda p: (0, 0)),
            pl.BlockSpec((256, 24), lambda p: (0, 0)),
            pl.BlockSpec((256, 24), lambda p: (0, 0)),
            pl.BlockSpec((256, 24), lambda p: (0, 0)),
            pl.BlockSpec((256, 24), lambda p: (0, 0)),
            pl.BlockSpec((1, 1), lambda p: (0, 0)),
        ],
        out_specs=pl.BlockSpec((1, 1), lambda p: (0, 0)),
        out_shape=jax.ShapeDtypeStruct((1, 1), jnp.float32),
        scratch_shapes=[pltpu.VMEM((8, 128), jnp.float32)],
    )(ai, an, w2bi, b2ti, w2bn, b2tn, abi, abn, dbi, dbn, db)


def _blockdiag(w):
    return jnp.kron(jnp.eye(8, dtype=jnp.float32), w)


@jax.jit
def kernel(x, edge_index_int, edge_index_nh, W1_int, b1_int, W1_nh, b1_nh,
           W2_int, b2_int, W2_nh, b2_nh, att_w, dense_W, dense_b):
    x_pad = jnp.pad(x, ((0, N_ACC - N), (0, D - x.shape[1])))
    zeros = jnp.zeros((N_ACC, D), jnp.float32)
    w1bi = _blockdiag(jnp.zeros((16, 16), jnp.float32).at[:11].set(W1_int))
    w1bn = _blockdiag(jnp.zeros((16, 16), jnp.float32).at[:11].set(W1_nh))
    b1ti = jnp.tile(b1_int, 8).reshape(1, 128)
    b1tn = jnp.tile(b1_nh, 8).reshape(1, 128)
    w2bi = _blockdiag(W2_int)
    w2bn = _blockdiag(W2_nh)
    b2ti = jnp.tile(b2_int, 8).reshape(1, 256)
    b2tn = jnp.tile(b2_nh, 8).reshape(1, 256)
    abi = _blockdiag(att_w[:32])
    abn = _blockdiag(att_w[32:])
    dwr = dense_W.reshape(3, 64)
    dbi = _blockdiag(dwr[:, :32].T)
    dbn = _blockdiag(dwr[:, 32:].T)
    db = dense_b.reshape(1, 1)

    agg1_i, agg1_n = _sc_seg_sum(
        x_pad, x_pad, edge_index_int[0], edge_index_int[1],
        edge_index_nh[0], edge_index_nh[1], zeros)

    h1_i, h1_n = _tc_mm_relu(agg1_i.reshape(PK, 128), agg1_n.reshape(PK, 128),
                             w1bi, b1ti, w1bn, b1tn)

    agg2_i, agg2_n = _sc_seg_sum(
        h1_i.reshape(N_ACC, D), h1_n.reshape(N_ACC, D),
        edge_index_int[0], edge_index_int[1],
        edge_index_nh[0], edge_index_nh[1], zeros)

    out = _tc_final(agg2_i.reshape(PK, 128), agg2_n.reshape(PK, 128),
                    w2bi, b2ti, w2bn, b2tn, abi, abn, dbi, dbn, db)
    return out.reshape(())


# speedup vs baseline: 47.7198x; 1.1914x over previous
import jax
import jax.numpy as jnp
from jax import lax
from jax.experimental import pallas as pl
from jax.experimental.pallas import tpu as pltpu
from jax.experimental.pallas import tpu_sc as plsc

N = 100000
E = 1600000
D = 16
NS = 16
ET = E // NS
W = 400
NWIN = ET // W
N_ACC = 100096
RT = N_ACC // NS
PK = N_ACC // 8
PKN = N // 8


def _core_seg_sum(sid, table, src, dst, out, acc, bufs):
    base = sid * ET
    srcs, dsts, rows, sem_i, sem_g, sem_s = bufs

    def idx_start(w, s):
        off = pl.multiple_of(base + w * W, 8)
        pltpu.async_copy(src.at[pl.ds(off, W)], srcs[s], sem_i[s])
        pltpu.async_copy(dst.at[pl.ds(off, W)], dsts[s], sem_i[s])

    def gather_start(s):
        pltpu.make_async_copy(src.at[pl.ds(0, W)], srcs[s], sem_i[s]).wait()
        pltpu.make_async_copy(dst.at[pl.ds(0, W)], dsts[s], sem_i[s]).wait()
        pltpu.async_copy(table.at[srcs[s]], rows[s], sem_g[s])

    def scatter_start(s):
        pltpu.make_async_copy(table.at[srcs[s]], rows[s], sem_g[s]).wait()
        pltpu.async_copy(rows[s], acc.at[dsts[s]], sem_s[s], add=True)

    def scatter_wait(s):
        pltpu.make_async_copy(rows[s], acc.at[dsts[s]], sem_s[s]).wait()

    idx_start(0, 0); gather_start(0)
    idx_start(1, 1); gather_start(1)
    idx_start(2, 2); scatter_start(0); gather_start(2)

    def step(k, carry):
        for j in range(3):
            w = 3 * k + j
            scatter_wait(j)
            idx_start(w, j)
            scatter_start((j + 1) % 3)
            gather_start(j)
        return carry

    lax.fori_loop(1, NWIN // 3, step, 0)
    scatter_wait(0); idx_start(NWIN - 1, 0)
    scatter_start(1); gather_start(0)
    scatter_wait(1); scatter_start(2)
    scatter_wait(2); scatter_start(0)
    scatter_wait(0)
    plsc.subcore_barrier()
    r0 = pl.multiple_of(sid * RT, 8)
    pltpu.sync_copy(acc.at[pl.ds(r0, RT)], out.at[pl.ds(r0, RT)])


def _seg_sum_body(table_int, table_nh, src_int, dst_int, src_nh, dst_nh,
                  zeros, out_int, out_nh, acc,
                  s0, d0, r0b, s1, d1, r1b, s2, d2, r2b,
                  si0, si1, si2, sg0, sg1, sg2, ss0, ss1, ss2):
    cid = lax.axis_index("c")
    sid = lax.axis_index("s")
    bufs = ([s0, s1, s2], [d0, d1, d2], [r0b, r1b, r2b],
            [si0, si1, si2], [sg0, sg1, sg2], [ss0, ss1, ss2])

    r0 = pl.multiple_of(sid * RT, 8)
    pltpu.sync_copy(zeros.at[pl.ds(r0, RT)], acc.at[pl.ds(r0, RT)])
    plsc.subcore_barrier()

    @pl.when(cid == 0)
    def _():
        _core_seg_sum(sid, table_int, src_int, dst_int, out_int, acc, bufs)

    @pl.when(cid == 1)
    def _():
        _core_seg_sum(sid, table_nh, src_nh, dst_nh, out_nh, acc, bufs)


def _sc_seg_sum(table_int, table_nh, src_int, dst_int, src_nh, dst_nh, zeros):
    mesh = plsc.VectorSubcoreMesh(core_axis_name="c", subcore_axis_name="s")
    return pl.kernel(
        _seg_sum_body,
        out_type=(jax.ShapeDtypeStruct((N_ACC, D), jnp.float32),
                  jax.ShapeDtypeStruct((N_ACC, D), jnp.float32)),
        mesh=mesh,
        scratch_types=(
            [pltpu.VMEM_SHARED((N_ACC, D), jnp.float32)]
            + [pltpu.VMEM((W,), jnp.int32), pltpu.VMEM((W,), jnp.int32),
               pltpu.VMEM((W, D), jnp.float32)] * 3
            + [pltpu.SemaphoreType.DMA] * 9
        ),
        compiler_params=pltpu.CompilerParams(use_tc_tiling_on_sc=False),
    )(table_int, table_nh, src_int, dst_int, src_nh, dst_nh, zeros)


CH = 9088
NCH = 11
TAIL = E - NS * NCH * CH


def _reformat_core(sid, edge, out_s, out_d, buf):
    base = sid * NCH * CH

    def chunk(c, carry):
        off = pl.multiple_of(base + c * CH, 128)
        pltpu.sync_copy(edge.at[:, pl.ds(off, CH)], buf)
        pltpu.sync_copy(buf.at[0], out_s.at[pl.ds(off, CH)])
        pltpu.sync_copy(buf.at[1], out_d.at[pl.ds(off, CH)])
        return carry

    lax.fori_loop(0, NCH, chunk, 0)

    @pl.when(sid == 0)
    def _():
        off = NS * NCH * CH
        pltpu.sync_copy(edge.at[:, pl.ds(off, TAIL)], buf.at[:, pl.ds(0, TAIL)])
        pltpu.sync_copy(buf.at[0, pl.ds(0, TAIL)], out_s.at[pl.ds(off, TAIL)])
        pltpu.sync_copy(buf.at[1, pl.ds(0, TAIL)], out_d.at[pl.ds(off, TAIL)])


def _reformat_body(ei, en, src_i, dst_i, src_n, dst_n, buf):
    cid = lax.axis_index("c")
    sid = lax.axis_index("s")

    @pl.when(cid == 0)
    def _():
        _reformat_core(sid, ei, src_i, dst_i, buf)

    @pl.when(cid == 1)
    def _():
        _reformat_core(sid, en, src_n, dst_n, buf)


def _sc_reformat(ei, en):
    mesh = plsc.VectorSubcoreMesh(core_axis_name="c", subcore_axis_name="s")
    return pl.kernel(
        _reformat_body,
        out_type=(jax.ShapeDtypeStruct((E,), jnp.int32),) * 4,
        mesh=mesh,
        scratch_types=[pltpu.VMEM((2, CH), jnp.int32)],
    )(ei, en)


BLK = 3128
NBLK = PK // BLK


def _mm_relu_body(a1_ref, a2_ref, w1_ref, b1_ref, w2_ref, b2_ref,
                  o1_ref, o2_ref):
    o1_ref[...] = jax.nn.relu(
        jnp.dot(a1_ref[...], w1_ref[...], preferred_element_type=jnp.float32,
                precision=jax.lax.Precision.HIGHEST) + b1_ref[...])
    o2_ref[...] = jax.nn.relu(
        jnp.dot(a2_ref[...], w2_ref[...], preferred_element_type=jnp.float32,
                precision=jax.lax.Precision.HIGHEST) + b2_ref[...])


def _tc_mm_relu(a1, a2, w1b, b1t, w2b, b2t):
    return pl.pallas_call(
        _mm_relu_body,
        grid=(NBLK,),
        in_specs=[
            pl.BlockSpec((BLK, 128), lambda p: (p, 0)),
            pl.BlockSpec((BLK, 128), lambda p: (p, 0)),
            pl.BlockSpec((128, 128), lambda p: (0, 0)),
            pl.BlockSpec((1, 128), lambda p: (0, 0)),
            pl.BlockSpec((128, 128), lambda p: (0, 0)),
            pl.BlockSpec((1, 128), lambda p: (0, 0)),
        ],
        out_specs=[
            pl.BlockSpec((BLK, 128), lambda p: (p, 0)),
            pl.BlockSpec((BLK, 128), lambda p: (p, 0)),
        ],
        out_shape=[jax.ShapeDtypeStruct((PK, 128), jnp.float32),
                   jax.ShapeDtypeStruct((PK, 128), jnp.float32)],
    )(a1, a2, w1b, b1t, w2b, b2t)


BLK2 = 3128
NBLK2 = PK // BLK2


def _final_body(ai_ref, an_ref, w2i_ref, b2i_ref, w2n_ref, b2n_ref,
                abi_ref, abn_ref, dbi_ref, dbn_ref, db_ref,
                o_ref, acc_ref):
    p = pl.program_id(0)

    @pl.when(p == 0)
    def _():
        acc_ref[...] = jnp.zeros_like(acc_ref)

    hp = jax.lax.Precision.HIGHEST
    h_i = jax.nn.relu(
        jnp.dot(ai_ref[...], w2i_ref[...], preferred_element_type=jnp.float32,
                precision=hp) + b2i_ref[...])
    h_n = jax.nn.relu(
        jnp.dot(an_ref[...], w2n_ref[...], preferred_element_type=jnp.float32,
                precision=hp) + b2n_ref[...])
    s = jnp.tanh(
        jnp.dot(h_i, abi_ref[...], preferred_element_type=jnp.float32,
                precision=hp)
        + jnp.dot(h_n, abn_ref[...], preferred_element_type=jnp.float32,
                  precision=hp))
    e = jnp.exp(s)
    proj = (jnp.dot(h_i, dbi_ref[...], preferred_element_type=jnp.float32,
                    precision=hp)
            + jnp.dot(h_n, dbn_ref[...], preferred_element_type=jnp.float32,
                      precision=hp))
    row = p * BLK2 + jax.lax.broadcasted_iota(jnp.int32, (BLK2, 1), 0)
    e = jnp.where(row < PKN, e, 0.0)
    acc_ref[0:1, 0:24] += jnp.sum(e, axis=0, keepdims=True)
    acc_ref[1:2, 0:24] += jnp.sum(e * proj, axis=0, keepdims=True)

    @pl.when(p == NBLK2 - 1)
    def _():
        den = jnp.zeros((1, 3), jnp.float32)
        num = jnp.zeros((1, 3), jnp.float32)
        for g in range(8):
            den += acc_ref[0:1, 3 * g:3 * g + 3]
            num += acc_ref[1:2, 3 * g:3 * g + 3]
        o_ref[...] = jnp.sum(num / den).reshape(1, 1) + db_ref[...]


def _tc_final(ai, an, w2bi, b2ti, w2bn, b2tn, abi, abn, dbi, dbn, db):
    return pl.pallas_call(
        _final_body,
        grid=(NBLK2,),
        in_specs=[
            pl.BlockSpec((BLK2, 128), lambda p: (p, 0)),
            pl.BlockSpec((BLK2, 128), lambda p: (p, 0)),
            pl.BlockSpec((128, 256), lambda p: (0, 0)),
            pl.BlockSpec((1, 256), lambda p: (0, 0)),
            pl.BlockSpec((128, 256), lambda p: (0, 0)),
            pl.BlockSpec((1, 256), lambda p: (0, 0)),
            pl.BlockSpec((256, 24), lambda p: (0, 0)),
            pl.BlockSpec((256, 24), lambda p: (0, 0)),
            pl.BlockSpec((256, 24), lambda p: (0, 0)),
            pl.BlockSpec((256, 24), lambda p: (0, 0)),
            pl.BlockSpec((1, 1), lambda p: (0, 0)),
        ],
        out_specs=pl.BlockSpec((1, 1), lambda p: (0, 0)),
        out_shape=jax.ShapeDtypeStruct((1, 1), jnp.float32),
        scratch_shapes=[pltpu.VMEM((8, 128), jnp.float32)],
    )(ai, an, w2bi, b2ti, w2bn, b2tn, abi, abn, dbi, dbn, db)


def _blockdiag(w):
    return jnp.kron(jnp.eye(8, dtype=jnp.float32), w)


@jax.jit
def kernel(x, edge_index_int, edge_index_nh, W1_int, b1_int, W1_nh, b1_nh,
           W2_int, b2_int, W2_nh, b2_nh, att_w, dense_W, dense_b):
    x_pad = jnp.pad(x, ((0, N_ACC - N), (0, D - x.shape[1])))
    zeros = jnp.zeros((N_ACC, D), jnp.float32)
    w1bi = _blockdiag(jnp.zeros((16, 16), jnp.float32).at[:11].set(W1_int))
    w1bn = _blockdiag(jnp.zeros((16, 16), jnp.float32).at[:11].set(W1_nh))
    b1ti = jnp.tile(b1_int, 8).reshape(1, 128)
    b1tn = jnp.tile(b1_nh, 8).reshape(1, 128)
    w2bi = _blockdiag(W2_int)
    w2bn = _blockdiag(W2_nh)
    b2ti = jnp.tile(b2_int, 8).reshape(1, 256)
    b2tn = jnp.tile(b2_nh, 8).reshape(1, 256)
    abi = _blockdiag(att_w[:32])
    abn = _blockdiag(att_w[32:])
    dwr = dense_W.reshape(3, 64)
    dbi = _blockdiag(dwr[:, :32].T)
    dbn = _blockdiag(dwr[:, 32:].T)
    db = dense_b.reshape(1, 1)

    src_i, dst_i, src_n, dst_n = _sc_reformat(edge_index_int, edge_index_nh)
    agg1_i, agg1_n = _sc_seg_sum(x_pad, x_pad, src_i, dst_i, src_n, dst_n,
                                 zeros)

    h1_i, h1_n = _tc_mm_relu(agg1_i.reshape(PK, 128), agg1_n.reshape(PK, 128),
                             w1bi, b1ti, w1bn, b1tn)

    agg2_i, agg2_n = _sc_seg_sum(
        h1_i.reshape(N_ACC, D), h1_n.reshape(N_ACC, D),
        src_i, dst_i, src_n, dst_n, zeros)

    out = _tc_final(agg2_i.reshape(PK, 128), agg2_n.reshape(PK, 128),
                    w2bi, b2ti, w2bn, b2tn, abi, abn, dbi, dbn, db)
    return out.reshape(())
